# baseline (speedup 1.0000x reference)
import functools

import jax
import jax.numpy as jnp
from jax.experimental import pallas as pl
from jax.experimental.pallas import tpu as pltpu

_N = 4096
_D = 64
_L = 128
_K = 256
_S = 1024.0
_BIG = _S * _S
_MARGIN = 1.0


def _triplet_block_kernel(bm, nb, e_all_ref, t_all_ref, out_ref,
                          a_ref, b_ref, sq_ref, cnt_ref, d0c_ref):
    i = pl.program_id(0)

    @pl.when(i == 0)
    def _pack():
        ef = e_all_ref[...]
        tj = t_all_ref[...]
        lanes = jax.lax.broadcasted_iota(jnp.int32, (1, _L), 1)
        oh = (tj == lanes).astype(jnp.float32)
        sq = jnp.sum(ef * ef, axis=1, keepdims=True)
        ohs = (oh * _S).astype(jnp.bfloat16)
        zpad = jnp.zeros((_N, _L - _D - 1), jnp.bfloat16)

        a_ref[:, 0:_D] = (ef * -2.0).astype(jnp.bfloat16)
        a_ref[:, _D:_D + 1] = jnp.ones((_N, 1), jnp.bfloat16)
        a_ref[:, _D + 1:_L] = zpad
        a_ref[:, _L:_K] = ohs

        b_ref[:, 0:_D] = ef.astype(jnp.bfloat16)
        b_ref[:, _D:_D + 1] = sq.astype(jnp.bfloat16)
        b_ref[:, _D + 1:_L] = zpad
        b_ref[:, _L:_K] = ohs

        sq_ref[...] = sq
        hist = jnp.sum(oh, axis=0, keepdims=True)
        cnt_ref[...] = jnp.sum(oh * hist, axis=1, keepdims=True)
        t0 = t_all_ref[0, 0]
        d0c_ref[...] = jnp.where(tj == t0, _BIG, 0.0)

    a_blk = a_ref[pl.ds(i * bm, bm), :]
    nchunk = 4
    w = _N // nchunk
    maxs, mins = [], []
    d0col = None
    for t in range(nchunk):
        ct = jax.lax.dot_general(
            a_blk, b_ref[pl.ds(t * w, w), :], (((1,), (1,)), ((), ())),
            preferred_element_type=jnp.float32)
        maxs.append(jnp.max(ct, axis=1))
        mins.append(jnp.min(ct, axis=1))
        if t == 0:
            d0col = ct[:, 0]

    sq_i = sq_ref[pl.ds(i * bm, bm), 0]
    pos_v = jnp.maximum(jnp.maximum(maxs[0], maxs[1]),
                        jnp.maximum(maxs[2], maxs[3])) + sq_i - _BIG
    neg_v = jnp.minimum(jnp.minimum(mins[0], mins[1]),
                        jnp.minimum(mins[2], mins[3])) + sq_i

    count = cnt_ref[pl.ds(i * bm, bm), 0]
    d0 = d0col + sq_i - d0c_ref[pl.ds(i * bm, bm), 0]
    ap = jnp.where(count > 1.5, pos_v, d0)
    an = jnp.where(count < _N - 0.5, neg_v, d0)

    losses = jnp.maximum(ap - an + _MARGIN, 0.0)
    s = jnp.sum(losses)

    @pl.when(i == 0)
    def _init_out():
        out_ref[...] = jnp.zeros((1, 1, 1), jnp.float32)

    acc = out_ref[0, 0, 0] + s
    out_ref[...] = jnp.where(i == nb - 1, acc / _N, acc).reshape(1, 1, 1)


def _triplet_mean_loss(embeddings, target, bm):
    nb = _N // bm
    tcol = target.astype(jnp.int32).reshape(_N, 1)
    out = pl.pallas_call(
        functools.partial(_triplet_block_kernel, bm, nb),
        grid=(nb,),
        in_specs=[
            pl.BlockSpec((_N, _D), lambda i: (0, 0)),
            pl.BlockSpec((_N, 1), lambda i: (0, 0)),
        ],
        out_specs=pl.BlockSpec((1, 1, 1), lambda i: (0, 0, 0)),
        out_shape=jax.ShapeDtypeStruct((1, 1, 1), jnp.float32),
        scratch_shapes=[
            pltpu.VMEM((_N, _K), jnp.bfloat16),
            pltpu.VMEM((_N, _K), jnp.bfloat16),
            pltpu.VMEM((_N, 1), jnp.float32),
            pltpu.VMEM((_N, 1), jnp.float32),
            pltpu.VMEM((_N, 1), jnp.float32),
        ],
    )(embeddings, tcol)
    return out.reshape(())


def kernel(embeddings, target):
    mean_loss = _triplet_mean_loss(embeddings, target, bm=512)
    return (mean_loss, _N)


# speedup vs baseline: 1.3041x; 1.3041x over previous
import jax
import jax.numpy as jnp
from jax.experimental import pallas as pl
from jax.experimental.pallas import tpu as pltpu

_N = 4096
_D = 64
_L = 128
_K = 256
_S = 1024.0
_BIG = _S * _S
_MARGIN = 1.0
_BM = 512


def _triplet_kernel(e_all_ref, t_all_ref, out_ref, a_ref, b_ref):
    ef = e_all_ref[...]
    tj = t_all_ref[...]
    lanes = jax.lax.broadcasted_iota(jnp.int32, (1, _L), 1)
    oh = (tj == lanes).astype(jnp.float32)
    sq = jnp.sum(ef * ef, axis=1, keepdims=True)
    ohs = (oh * _S).astype(jnp.bfloat16)
    zpad = jnp.zeros((_N, _L - _D - 1), jnp.bfloat16)

    a_ref[:, 0:_D] = (ef * -2.0).astype(jnp.bfloat16)
    a_ref[:, _D:_D + 1] = jnp.ones((_N, 1), jnp.bfloat16)
    a_ref[:, _D + 1:_L] = zpad
    a_ref[:, _L:_K] = ohs

    b_ref[:, 0:_D] = ef.astype(jnp.bfloat16)
    b_ref[:, _D:_D + 1] = sq.astype(jnp.bfloat16)
    b_ref[:, _D + 1:_L] = zpad
    b_ref[:, _L:_K] = ohs

    hist = jnp.sum(oh, axis=0, keepdims=True)
    cnt = jnp.sum(oh * hist, axis=1, keepdims=True)
    t0 = t_all_ref[0, 0]
    d0corr = jnp.where(tj == t0, _BIG, 0.0)

    nb = _N // _BM
    total = jnp.zeros((), jnp.float32)
    for blk in range(nb):
        rows = pl.ds(blk * _BM, _BM)
        c = jax.lax.dot_general(
            a_ref[rows, :], b_ref[...], (((1,), (1,)), ((), ())),
            preferred_element_type=jnp.float32)
        sq_i = sq[blk * _BM:(blk + 1) * _BM, 0]
        pos_v = jnp.max(c, axis=1) + sq_i - _BIG
        neg_v = jnp.min(c, axis=1) + sq_i
        d0 = c[:, 0] + sq_i - d0corr[blk * _BM:(blk + 1) * _BM, 0]
        count = cnt[blk * _BM:(blk + 1) * _BM, 0]
        ap = jnp.where(count > 1.5, pos_v, d0)
        an = jnp.where(count < _N - 0.5, neg_v, d0)
        total = total + jnp.sum(jnp.maximum(ap - an + _MARGIN, 0.0))

    out_ref[...] = (total / _N).reshape(1, 1, 1)


def _triplet_mean_loss(embeddings, target):
    tcol = target.astype(jnp.int32).reshape(_N, 1)
    out = pl.pallas_call(
        _triplet_kernel,
        in_specs=[
            pl.BlockSpec((_N, _D), lambda: (0, 0)),
            pl.BlockSpec((_N, 1), lambda: (0, 0)),
        ],
        out_specs=pl.BlockSpec((1, 1, 1), lambda: (0, 0, 0)),
        out_shape=jax.ShapeDtypeStruct((1, 1, 1), jnp.float32),
        scratch_shapes=[
            pltpu.VMEM((_N, _K), jnp.bfloat16),
            pltpu.VMEM((_N, _K), jnp.bfloat16),
        ],
    )(embeddings, tcol)
    return out.reshape(())


def kernel(embeddings, target):
    return (_triplet_mean_loss(embeddings, target), _N)
